# baseline (speedup 1.0000x reference)
import functools

import jax
import jax.numpy as jnp
from jax import lax
from jax.experimental import pallas as pl
from jax.experimental.pallas import tpu as pltpu
from jax.experimental.pallas import tpu_sc as plsc

VOCAB = 1000000
DIM = 64
B = 16384
L = 50
H1 = 128
H2 = 64
OUT = 10

NC = 2
NS = 16
NW = NC * NS
BPW = B // NW
CB = 16
NCHUNK = BPW // CB
IDX_PER_CHUNK = CB * L
GI = 80
G = IDX_PER_CHUNK // GI
NLG = DIM // 16
KU = 10


def _pool_body(xf_hbm, table_hbm, out_hbm,
               idx0, idx1, rows0, rows1, acc0, acc1,
               isem, gsem0, gsem1, osem0, osem1):
    w = lax.axis_index("s") * NC + lax.axis_index("c")
    xbase = w * (BPW * L)
    obase = w * BPW
    idx_v = (idx0, idx1)
    rows_v = (rows0, rows1)
    acc_v = (acc0, acc1)
    gsem = (gsem0, gsem1)
    osem = (osem0, osem1)

    def issue_idx(c, p):
        pltpu.async_copy(
            xf_hbm.at[pl.ds(xbase + c * IDX_PER_CHUNK, IDX_PER_CHUNK)],
            idx_v[p], isem)

    def wait_idx(p):
        pltpu.make_async_copy(
            xf_hbm.at[pl.ds(0, IDX_PER_CHUNK)], idx_v[p], isem).wait()

    def permute_idx(p):
        ref = idx_v[p]
        for i in range(IDX_PER_CHUNK // 16):
            v = ref[pl.ds(i * 16, 16)]
            ref[pl.ds(i * 16, 16)] = 2 * v - jnp.where(
                v >= (VOCAB // 2), VOCAB - 1, 0)

    def issue_gathers(p):
        for t in range(G):
            pltpu.async_copy(
                table_hbm.at[idx_v[p].at[pl.ds(t * GI, GI)]],
                rows_v[p].at[pl.ds(t * GI, GI), :], gsem[p])

    def drain_gathers(p):
        for t in range(G):
            pltpu.make_async_copy(
                table_hbm.at[idx_v[p].at[pl.ds(t * GI, GI)]],
                rows_v[p].at[pl.ds(t * GI, GI), :], gsem[p]).wait()

    def issue_out(c, p):
        pltpu.async_copy(
            acc_v[p], out_hbm.at[pl.ds(obase + c * CB, CB), :], osem[p])

    def drain_out(p):
        pltpu.make_async_copy(
            acc_v[p], out_hbm.at[pl.ds(0, CB), :], osem[p]).wait()

    def reduce(p):
        rv = rows_v[p]
        for b in range(CB):
            def red(kk, accs):
                base = b * L + kk * KU
                for u in range(KU):
                    accs = tuple(
                        accs[g] + rv[base + u, pl.ds(g * 16, 16)]
                        for g in range(NLG)
                    )
                return accs

            accs = lax.fori_loop(
                0, L // KU, red,
                tuple(jnp.zeros((16,), jnp.float32) for _ in range(NLG)),
            )
            for g in range(NLG):
                acc_v[p][b, pl.ds(g * 16, 16)] = accs[g]

    issue_idx(0, 0)
    wait_idx(0)
    permute_idx(0)
    issue_gathers(0)
    issue_idx(1, 1)

    def body(jj, carry):
        for p in (0, 1):
            j = 2 * jj + p

            @pl.when(j + 1 < NCHUNK)
            def _():
                wait_idx(1 - p)
                permute_idx(1 - p)
                issue_gathers(1 - p)

            drain_gathers(p)

            @pl.when(j + 2 < NCHUNK)
            def _():
                issue_idx(j + 2, p)

            @pl.when(jj >= 1)
            def _():
                drain_out(p)

            reduce(p)
            issue_out(j, p)
        return carry

    lax.fori_loop(0, NCHUNK // 2, body, 0)
    drain_out(0)
    drain_out(1)


def _make_pool():
    mesh = plsc.VectorSubcoreMesh(core_axis_name="c", subcore_axis_name="s")
    return pl.kernel(
        _pool_body,
        mesh=mesh,
        out_type=jax.ShapeDtypeStruct((B, DIM), jnp.float32),
        scratch_types=[
            pltpu.VMEM((IDX_PER_CHUNK,), jnp.int32),
            pltpu.VMEM((IDX_PER_CHUNK,), jnp.int32),
            pltpu.VMEM((IDX_PER_CHUNK, DIM), jnp.float32),
            pltpu.VMEM((IDX_PER_CHUNK, DIM), jnp.float32),
            pltpu.VMEM((CB, DIM), jnp.float32),
            pltpu.VMEM((CB, DIM), jnp.float32),
            pltpu.SemaphoreType.DMA,
            pltpu.SemaphoreType.DMA,
            pltpu.SemaphoreType.DMA,
            pltpu.SemaphoreType.DMA,
            pltpu.SemaphoreType.DMA,
        ],
        compiler_params=pltpu.CompilerParams(use_tc_tiling_on_sc=False),
    )


def _relayout_kernel(t1_ref, t2_ref, o_ref):
    o_ref[...] = jnp.concatenate([t1_ref[...], t2_ref[...]], axis=1)


def _relayout(table):
    bs = 4000
    half = VOCAB // 2
    grid = half // bs
    out = pl.pallas_call(
        _relayout_kernel,
        grid=(grid,),
        in_specs=[
            pl.BlockSpec((bs, DIM), lambda i: (i, 0)),
            pl.BlockSpec((bs, DIM), lambda i: (i + half // bs, 0)),
        ],
        out_specs=pl.BlockSpec((bs, 2 * DIM), lambda i: (i, 0)),
        out_shape=jax.ShapeDtypeStruct((half, 2 * DIM), jnp.float32),
    )(table, table)
    return out.reshape(VOCAB, DIM)


def _mlp_kernel(sums_ref, len_ref, w1_ref, b1_ref, w2_ref, b2_ref, w3_ref,
                b3_ref, out_ref):
    rep = sums_ref[...] / len_ref[...]
    h1 = jnp.tanh(
        jnp.dot(rep, w1_ref[...], preferred_element_type=jnp.float32)
        + b1_ref[...]
    )
    z2 = (
        jnp.dot(h1, w2_ref[...], preferred_element_type=jnp.float32)
        + b2_ref[...]
    )
    h2 = jnp.where(z2 >= 0, z2, 0.01 * z2)
    out_ref[...] = (
        jnp.dot(h2, w3_ref[...], preferred_element_type=jnp.float32)
        + b3_ref[...]
    )


def _mlp(sums, lens_f, W1, b1, W2, b2, W3, b3):
    bs = 2048
    grid = B // bs
    full = lambda shape: pl.BlockSpec(shape, lambda i: (0, 0))
    return pl.pallas_call(
        _mlp_kernel,
        grid=(grid,),
        in_specs=[
            pl.BlockSpec((bs, DIM), lambda i: (i, 0)),
            pl.BlockSpec((bs, 1), lambda i: (i, 0)),
            full((DIM, H1)),
            full((1, H1)),
            full((H1, H2)),
            full((1, H2)),
            full((H2, OUT)),
            full((1, OUT)),
        ],
        out_specs=pl.BlockSpec((bs, OUT), lambda i: (i, 0)),
        out_shape=jax.ShapeDtypeStruct((B, OUT), jnp.float32),
    )(sums, lens_f, W1, b1, W2, b2, W3, b3)


@jax.jit
def kernel(x, lengths, table, W1, b1, W2, b2, W3, b3):
    xf = x.reshape(B * L)
    sums = _make_pool()(xf, _relayout(table))
    lens_f = lengths.astype(jnp.float32).reshape(B, 1)
    return _mlp(
        sums, lens_f, W1,
        b1.reshape(1, H1), W2, b2.reshape(1, H2), W3, b3.reshape(1, OUT),
    )


# speedup vs baseline: 1.5005x; 1.5005x over previous
import functools

import jax
import jax.numpy as jnp
from jax import lax
from jax.experimental import pallas as pl
from jax.experimental.pallas import tpu as pltpu
from jax.experimental.pallas import tpu_sc as plsc

VOCAB = 1000000
DIM = 64
B = 16384
L = 50
H1 = 128
H2 = 64
OUT = 10

NC = 2
NS = 16
NW = NC * NS
BPW = B // NW
CB = 16
NCHUNK = BPW // CB
IDX_PER_CHUNK = CB * L
GI = 80
G = IDX_PER_CHUNK // GI
NLG = DIM // 16
KU = 10


def _pool_body(xf_hbm, table_hbm, out_hbm,
               idx0, idx1, rows0, rows1, acc0, acc1,
               isem, gsem0, gsem1, osem0, osem1):
    w = lax.axis_index("s") * NC + lax.axis_index("c")
    xbase = w * (BPW * L)
    obase = w * BPW
    idx_v = (idx0, idx1)
    rows_v = (rows0, rows1)
    acc_v = (acc0, acc1)
    gsem = (gsem0, gsem1)
    osem = (osem0, osem1)

    def issue_idx(c, p):
        pltpu.async_copy(
            xf_hbm.at[pl.ds(xbase + c * IDX_PER_CHUNK, IDX_PER_CHUNK)],
            idx_v[p], isem)

    def wait_idx(p):
        pltpu.make_async_copy(
            xf_hbm.at[pl.ds(0, IDX_PER_CHUNK)], idx_v[p], isem).wait()

    def permute_idx(p):
        ref = idx_v[p]
        for i in range(IDX_PER_CHUNK // 16):
            v = ref[pl.ds(i * 16, 16)]
            main = (
                (v & jnp.int32(~4095))
                + 2 * (v & jnp.int32(2047))
                + ((v >> 11) & jnp.int32(1))
            )
            w = v - jnp.int32(MAIN)
            tail = jnp.int32(MAIN) + (
                (w & jnp.int32(~127))
                + 2 * (w & jnp.int32(63))
                + ((w >> 6) & jnp.int32(1))
            )
            ref[pl.ds(i * 16, 16)] = jnp.where(v < MAIN, main, tail)

    def issue_gathers(p):
        for t in range(G):
            pltpu.async_copy(
                table_hbm.at[idx_v[p].at[pl.ds(t * GI, GI)]],
                rows_v[p].at[pl.ds(t * GI, GI), :], gsem[p])

    def drain_gathers(p):
        for t in range(G):
            pltpu.make_async_copy(
                table_hbm.at[idx_v[p].at[pl.ds(t * GI, GI)]],
                rows_v[p].at[pl.ds(t * GI, GI), :], gsem[p]).wait()

    def issue_out(c, p):
        pltpu.async_copy(
            acc_v[p], out_hbm.at[pl.ds(obase + c * CB, CB), :], osem[p])

    def drain_out(p):
        pltpu.make_async_copy(
            acc_v[p], out_hbm.at[pl.ds(0, CB), :], osem[p]).wait()

    def reduce(p):
        rv = rows_v[p]
        for b in range(CB):
            def red(kk, accs):
                base = b * L + kk * KU
                for u in range(KU):
                    accs = tuple(
                        accs[g] + rv[base + u, pl.ds(g * 16, 16)]
                        for g in range(NLG)
                    )
                return accs

            accs = lax.fori_loop(
                0, L // KU, red,
                tuple(jnp.zeros((16,), jnp.float32) for _ in range(NLG)),
            )
            for g in range(NLG):
                acc_v[p][b, pl.ds(g * 16, 16)] = accs[g]

    issue_idx(0, 0)
    wait_idx(0)
    permute_idx(0)
    issue_gathers(0)
    issue_idx(1, 1)

    def body(jj, carry):
        for p in (0, 1):
            j = 2 * jj + p

            @pl.when(j + 1 < NCHUNK)
            def _():
                wait_idx(1 - p)
                permute_idx(1 - p)
                issue_gathers(1 - p)

            drain_gathers(p)

            @pl.when(j + 2 < NCHUNK)
            def _():
                issue_idx(j + 2, p)

            @pl.when(jj >= 1)
            def _():
                drain_out(p)

            reduce(p)
            issue_out(j, p)
        return carry

    lax.fori_loop(0, NCHUNK // 2, body, 0)
    drain_out(0)
    drain_out(1)


def _make_pool():
    mesh = plsc.VectorSubcoreMesh(core_axis_name="c", subcore_axis_name="s")
    return pl.kernel(
        _pool_body,
        mesh=mesh,
        out_type=jax.ShapeDtypeStruct((B, DIM), jnp.float32),
        scratch_types=[
            pltpu.VMEM((IDX_PER_CHUNK,), jnp.int32),
            pltpu.VMEM((IDX_PER_CHUNK,), jnp.int32),
            pltpu.VMEM((IDX_PER_CHUNK, DIM), jnp.float32),
            pltpu.VMEM((IDX_PER_CHUNK, DIM), jnp.float32),
            pltpu.VMEM((CB, DIM), jnp.float32),
            pltpu.VMEM((CB, DIM), jnp.float32),
            pltpu.SemaphoreType.DMA,
            pltpu.SemaphoreType.DMA,
            pltpu.SemaphoreType.DMA,
            pltpu.SemaphoreType.DMA,
            pltpu.SemaphoreType.DMA,
        ],
        compiler_params=pltpu.CompilerParams(use_tc_tiling_on_sc=False),
    )


MAIN = 999424
TAILB = 5
VOCABP = MAIN + TAILB * 128


def _relayout_main_kernel(t_ref, o_ref):
    tr = jnp.swapaxes(t_ref[...], 0, 1)
    o_ref[...] = jnp.concatenate([tr[0:2048, :], tr[2048:4096, :]], axis=1)


def _relayout_tail_kernel(alias_ref, t_ref, o_ref):
    del alias_ref
    tr = jnp.swapaxes(t_ref[...], 0, 1)
    o_ref[...] = jnp.concatenate([tr[0:64, :], tr[64:128, :]], axis=1)


def _relayout(table_t):
    main = pl.pallas_call(
        _relayout_main_kernel,
        grid=(244,),
        in_specs=[pl.BlockSpec((DIM, 4096), lambda i: (0, i))],
        out_specs=pl.BlockSpec((2048, 2 * DIM), lambda i: (i, 0)),
        out_shape=jax.ShapeDtypeStruct((VOCABP // 2, 2 * DIM), jnp.float32),
    )(table_t)
    full = pl.pallas_call(
        _relayout_tail_kernel,
        grid=(TAILB,),
        in_specs=[
            pl.BlockSpec(memory_space=pltpu.MemorySpace.HBM),
            pl.BlockSpec((DIM, 128), lambda i: (0, MAIN // 128 + i)),
        ],
        out_specs=pl.BlockSpec((64, 2 * DIM), lambda i: (MAIN // 128 + i, 0)),
        out_shape=jax.ShapeDtypeStruct((VOCABP // 2, 2 * DIM), jnp.float32),
        input_output_aliases={0: 0},
    )(main, table_t)
    return full


def _mlp_kernel(sums_ref, len_ref, w1_ref, b1_ref, w2_ref, b2_ref, w3_ref,
                b3_ref, out_ref):
    rep = sums_ref[...] / len_ref[...]
    h1 = jnp.tanh(
        jnp.dot(rep, w1_ref[...], preferred_element_type=jnp.float32)
        + b1_ref[...]
    )
    z2 = (
        jnp.dot(h1, w2_ref[...], preferred_element_type=jnp.float32)
        + b2_ref[...]
    )
    h2 = jnp.where(z2 >= 0, z2, 0.01 * z2)
    out_ref[...] = (
        jnp.dot(h2, w3_ref[...], preferred_element_type=jnp.float32)
        + b3_ref[...]
    )


def _mlp(sums, lens_f, W1, b1, W2, b2, W3, b3):
    bs = 2048
    grid = B // bs
    full = lambda shape: pl.BlockSpec(shape, lambda i: (0, 0))
    return pl.pallas_call(
        _mlp_kernel,
        grid=(grid,),
        in_specs=[
            pl.BlockSpec((bs, DIM), lambda i: (i, 0)),
            pl.BlockSpec((bs, 1), lambda i: (i, 0)),
            full((DIM, H1)),
            full((1, H1)),
            full((H1, H2)),
            full((1, H2)),
            full((H2, OUT)),
            full((1, OUT)),
        ],
        out_specs=pl.BlockSpec((bs, OUT), lambda i: (i, 0)),
        out_shape=jax.ShapeDtypeStruct((B, OUT), jnp.float32),
    )(sums, lens_f, W1, b1, W2, b2, W3, b3)


@jax.jit
def kernel(x, lengths, table, W1, b1, W2, b2, W3, b3):
    xf = x.reshape(B * L)
    table_lin = _relayout(table.T).reshape(VOCABP, DIM)
    sums = _make_pool()(xf, table_lin)
    lens_f = lengths.astype(jnp.float32).reshape(B, 1)
    return _mlp(
        sums, lens_f, W1,
        b1.reshape(1, H1), W2, b2.reshape(1, H2), W3, b3.reshape(1, OUT),
    )


# speedup vs baseline: 1.7345x; 1.1560x over previous
import functools

import jax
import jax.numpy as jnp
from jax import lax
from jax.experimental import pallas as pl
from jax.experimental.pallas import tpu as pltpu
from jax.experimental.pallas import tpu_sc as plsc

VOCAB = 1000000
DIM = 64
B = 16384
L = 50
H1 = 128
H2 = 64
OUT = 10

NC = 2
NS = 16
NW = NC * NS
BPW = B // NW
CB = 16
NCHUNK = BPW // CB
IDX_PER_CHUNK = CB * L
GI = 80
G = IDX_PER_CHUNK // GI
NLG = DIM // 16
KU = 10


def _pool_body(xf_hbm, table_hbm, out_hbm,
               idx0, idx1, rows0, rows1, acc0, acc1,
               isem, gsem0, gsem1, osem0, osem1):
    w = lax.axis_index("s") * NC + lax.axis_index("c")
    xbase = w * (BPW * L)
    obase = w * BPW
    idx_v = (idx0, idx1)
    rows_v = (rows0, rows1)
    acc_v = (acc0, acc1)
    gsem = (gsem0, gsem1)
    osem = (osem0, osem1)

    def issue_idx(c, p):
        pltpu.async_copy(
            xf_hbm.at[pl.ds(xbase + c * IDX_PER_CHUNK, IDX_PER_CHUNK)],
            idx_v[p], isem)

    def wait_idx(p):
        pltpu.make_async_copy(
            xf_hbm.at[pl.ds(0, IDX_PER_CHUNK)], idx_v[p], isem).wait()

    def permute_idx(p):
        ref = idx_v[p]
        for i in range(IDX_PER_CHUNK // 16):
            v = ref[pl.ds(i * 16, 16)]
            main = (
                (v & jnp.int32(~(MBS - 1)))
                + 2 * (v & jnp.int32(MBS // 2 - 1))
                + ((v >> 12) & jnp.int32(1))
            )
            w = v - jnp.int32(MAIN)
            tail = jnp.int32(MAIN) + (
                (w & jnp.int32(~127))
                + 2 * (w & jnp.int32(63))
                + ((w >> 6) & jnp.int32(1))
            )
            ref[pl.ds(i * 16, 16)] = jnp.where(v < MAIN, main, tail)

    def issue_gathers(p):
        for t in range(G):
            pltpu.async_copy(
                table_hbm.at[idx_v[p].at[pl.ds(t * GI, GI)]],
                rows_v[p].at[pl.ds(t * GI, GI), :], gsem[p])

    def drain_gathers(p):
        for t in range(G):
            pltpu.make_async_copy(
                table_hbm.at[idx_v[p].at[pl.ds(t * GI, GI)]],
                rows_v[p].at[pl.ds(t * GI, GI), :], gsem[p]).wait()

    def issue_out(c, p):
        pltpu.async_copy(
            acc_v[p], out_hbm.at[pl.ds(obase + c * CB, CB), :], osem[p])

    def drain_out(p):
        pltpu.make_async_copy(
            acc_v[p], out_hbm.at[pl.ds(0, CB), :], osem[p]).wait()

    def reduce(p):
        rv = rows_v[p]
        for b in range(CB):
            def red(kk, accs):
                base = b * L + kk * KU
                for u in range(KU):
                    accs = tuple(
                        accs[g] + rv[base + u, pl.ds(g * 16, 16)]
                        for g in range(NLG)
                    )
                return accs

            accs = lax.fori_loop(
                0, L // KU, red,
                tuple(jnp.zeros((16,), jnp.float32) for _ in range(NLG)),
            )
            for g in range(NLG):
                acc_v[p][b, pl.ds(g * 16, 16)] = accs[g]

    issue_idx(0, 0)
    wait_idx(0)
    permute_idx(0)
    issue_gathers(0)
    issue_idx(1, 1)

    def body(jj, carry):
        for p in (0, 1):
            j = 2 * jj + p

            @pl.when(j + 1 < NCHUNK)
            def _():
                wait_idx(1 - p)
                permute_idx(1 - p)
                issue_gathers(1 - p)

            drain_gathers(p)

            @pl.when(j + 2 < NCHUNK)
            def _():
                issue_idx(j + 2, p)

            @pl.when(jj >= 1)
            def _():
                drain_out(p)

            reduce(p)
            issue_out(j, p)
        return carry

    lax.fori_loop(0, NCHUNK // 2, body, 0)
    drain_out(0)
    drain_out(1)


def _make_pool():
    mesh = plsc.VectorSubcoreMesh(core_axis_name="c", subcore_axis_name="s")
    return pl.kernel(
        _pool_body,
        mesh=mesh,
        out_type=jax.ShapeDtypeStruct((B, DIM), jnp.float32),
        scratch_types=[
            pltpu.VMEM((IDX_PER_CHUNK,), jnp.int32),
            pltpu.VMEM((IDX_PER_CHUNK,), jnp.int32),
            pltpu.VMEM((IDX_PER_CHUNK, DIM), jnp.float32),
            pltpu.VMEM((IDX_PER_CHUNK, DIM), jnp.float32),
            pltpu.VMEM((CB, DIM), jnp.float32),
            pltpu.VMEM((CB, DIM), jnp.float32),
            pltpu.SemaphoreType.DMA,
            pltpu.SemaphoreType.DMA,
            pltpu.SemaphoreType.DMA,
            pltpu.SemaphoreType.DMA,
            pltpu.SemaphoreType.DMA,
        ],
        compiler_params=pltpu.CompilerParams(use_tc_tiling_on_sc=False),
    )


MAIN = 999424
TAILB = 5
VOCABP = MAIN + TAILB * 128


MBS = 8192


def _relayout_main_kernel(t_ref, o_ref):
    tr = jnp.swapaxes(t_ref[...], 0, 1)
    o_ref[...] = jnp.concatenate(
        [tr[0:MBS // 2, :], tr[MBS // 2:MBS, :]], axis=1)


def _relayout_tail_kernel(alias_ref, t_ref, o_ref):
    del alias_ref
    tr = jnp.swapaxes(t_ref[...], 0, 1)
    o_ref[...] = jnp.concatenate([tr[0:64, :], tr[64:128, :]], axis=1)


def _relayout(table_t):
    main = pl.pallas_call(
        _relayout_main_kernel,
        grid=(MAIN // MBS,),
        in_specs=[pl.BlockSpec((DIM, MBS), lambda i: (0, i))],
        out_specs=pl.BlockSpec((MBS // 2, 2 * DIM), lambda i: (i, 0)),
        out_shape=jax.ShapeDtypeStruct((VOCABP // 2, 2 * DIM), jnp.float32),
    )(table_t)
    full = pl.pallas_call(
        _relayout_tail_kernel,
        grid=(TAILB,),
        in_specs=[
            pl.BlockSpec(memory_space=pltpu.MemorySpace.HBM),
            pl.BlockSpec((DIM, 128), lambda i: (0, MAIN // 128 + i)),
        ],
        out_specs=pl.BlockSpec((64, 2 * DIM), lambda i: (MAIN // 128 + i, 0)),
        out_shape=jax.ShapeDtypeStruct((VOCABP // 2, 2 * DIM), jnp.float32),
        input_output_aliases={0: 0},
    )(main, table_t)
    return full


def _mlp_kernel(sums_ref, len_ref, w1_ref, b1_ref, w2_ref, b2_ref, w3_ref,
                b3_ref, out_ref):
    rep = sums_ref[...] / len_ref[...]
    h1 = jnp.tanh(
        jnp.dot(rep, w1_ref[...], preferred_element_type=jnp.float32)
        + b1_ref[...]
    )
    z2 = (
        jnp.dot(h1, w2_ref[...], preferred_element_type=jnp.float32)
        + b2_ref[...]
    )
    h2 = jnp.where(z2 >= 0, z2, 0.01 * z2)
    out_ref[...] = (
        jnp.dot(h2, w3_ref[...], preferred_element_type=jnp.float32)
        + b3_ref[...]
    )


def _mlp(sums, lens_f, W1, b1, W2, b2, W3, b3):
    bs = 2048
    grid = B // bs
    full = lambda shape: pl.BlockSpec(shape, lambda i: (0, 0))
    return pl.pallas_call(
        _mlp_kernel,
        grid=(grid,),
        in_specs=[
            pl.BlockSpec((bs, DIM), lambda i: (i, 0)),
            pl.BlockSpec((bs, 1), lambda i: (i, 0)),
            full((DIM, H1)),
            full((1, H1)),
            full((H1, H2)),
            full((1, H2)),
            full((H2, OUT)),
            full((1, OUT)),
        ],
        out_specs=pl.BlockSpec((bs, OUT), lambda i: (i, 0)),
        out_shape=jax.ShapeDtypeStruct((B, OUT), jnp.float32),
    )(sums, lens_f, W1, b1, W2, b2, W3, b3)


@jax.jit
def kernel(x, lengths, table, W1, b1, W2, b2, W3, b3):
    xf = x.reshape(B * L)
    table_lin = _relayout(table.T).reshape(VOCABP, DIM)
    sums = _make_pool()(xf, table_lin)
    lens_f = lengths.astype(jnp.float32).reshape(B, 1)
    return _mlp(
        sums, lens_f, W1,
        b1.reshape(1, H1), W2, b2.reshape(1, H2), W3, b3.reshape(1, OUT),
    )


# speedup vs baseline: 1.8993x; 1.0950x over previous
import functools

import jax
import jax.numpy as jnp
from jax import lax
from jax.experimental import pallas as pl
from jax.experimental.pallas import tpu as pltpu
from jax.experimental.pallas import tpu_sc as plsc

VOCAB = 1000000
DIM = 64
B = 16384
L = 50
H1 = 128
H2 = 64
OUT = 10

NC = 2
NS = 16
NW = NC * NS
BPW = B // NW
CB = 16
NCHUNK = BPW // CB
IDX_PER_CHUNK = CB * L
GI = 80
G = IDX_PER_CHUNK // GI
NLG = DIM // 16
KU = 10


def _pool_body(xf_hbm, table_hbm, out_hbm,
               idx0, idx1, rows0, rows1, acc0, acc1,
               isem, gsem0, gsem1, osem0, osem1):
    w = lax.axis_index("s") * NC + lax.axis_index("c")
    xbase = w * (BPW * L)
    obase = w * BPW
    idx_v = (idx0, idx1)
    rows_v = (rows0, rows1)
    acc_v = (acc0, acc1)
    gsem = (gsem0, gsem1)
    osem = (osem0, osem1)

    def issue_idx(c, p):
        pltpu.async_copy(
            xf_hbm.at[pl.ds(xbase + c * IDX_PER_CHUNK, IDX_PER_CHUNK)],
            idx_v[p], isem)

    def wait_idx(p):
        pltpu.make_async_copy(
            xf_hbm.at[pl.ds(0, IDX_PER_CHUNK)], idx_v[p], isem).wait()

    def permute_idx(p):
        ref = idx_v[p]
        for i in range(IDX_PER_CHUNK // 16):
            v = ref[pl.ds(i * 16, 16)]
            main = (
                (v & jnp.int32(~(MBS - 1)))
                + 2 * (v & jnp.int32(MBS // 2 - 1))
                + ((v >> 13) & jnp.int32(1))
            )
            w = v - jnp.int32(MAIN)
            tail = jnp.int32(MAIN) + (
                (w & jnp.int32(~127))
                + 2 * (w & jnp.int32(63))
                + ((w >> 6) & jnp.int32(1))
            )
            ref[pl.ds(i * 16, 16)] = jnp.where(v < MAIN, main, tail)

    def issue_gathers(p):
        for t in range(G):
            pltpu.async_copy(
                table_hbm.at[idx_v[p].at[pl.ds(t * GI, GI)]],
                rows_v[p].at[pl.ds(t * GI, GI), :], gsem[p])

    def drain_gathers(p):
        for t in range(G):
            pltpu.make_async_copy(
                table_hbm.at[idx_v[p].at[pl.ds(t * GI, GI)]],
                rows_v[p].at[pl.ds(t * GI, GI), :], gsem[p]).wait()

    def issue_out(c, p):
        pltpu.async_copy(
            acc_v[p], out_hbm.at[pl.ds(obase + c * CB, CB), :], osem[p])

    def drain_out(p):
        pltpu.make_async_copy(
            acc_v[p], out_hbm.at[pl.ds(0, CB), :], osem[p]).wait()

    def reduce(p):
        rv = rows_v[p]
        for b in range(CB):
            def red(kk, accs):
                base = b * L + kk * KU
                for u in range(KU):
                    accs = tuple(
                        accs[g] + rv[base + u, pl.ds(g * 16, 16)]
                        for g in range(NLG)
                    )
                return accs

            accs = lax.fori_loop(
                0, L // KU, red,
                tuple(jnp.zeros((16,), jnp.float32) for _ in range(NLG)),
            )
            for g in range(NLG):
                acc_v[p][b, pl.ds(g * 16, 16)] = accs[g]

    issue_idx(0, 0)
    wait_idx(0)
    permute_idx(0)
    issue_gathers(0)
    issue_idx(1, 1)

    def body(jj, carry):
        for p in (0, 1):
            j = 2 * jj + p

            @pl.when(j + 1 < NCHUNK)
            def _():
                wait_idx(1 - p)
                permute_idx(1 - p)
                issue_gathers(1 - p)

            drain_gathers(p)

            @pl.when(j + 2 < NCHUNK)
            def _():
                issue_idx(j + 2, p)

            @pl.when(jj >= 1)
            def _():
                drain_out(p)

            reduce(p)
            issue_out(j, p)
        return carry

    lax.fori_loop(0, NCHUNK // 2, body, 0)
    drain_out(0)
    drain_out(1)


def _make_pool():
    mesh = plsc.VectorSubcoreMesh(core_axis_name="c", subcore_axis_name="s")
    return pl.kernel(
        _pool_body,
        mesh=mesh,
        out_type=jax.ShapeDtypeStruct((B, DIM), jnp.float32),
        scratch_types=[
            pltpu.VMEM((IDX_PER_CHUNK,), jnp.int32),
            pltpu.VMEM((IDX_PER_CHUNK,), jnp.int32),
            pltpu.VMEM((IDX_PER_CHUNK, DIM), jnp.float32),
            pltpu.VMEM((IDX_PER_CHUNK, DIM), jnp.float32),
            pltpu.VMEM((CB, DIM), jnp.float32),
            pltpu.VMEM((CB, DIM), jnp.float32),
            pltpu.SemaphoreType.DMA,
            pltpu.SemaphoreType.DMA,
            pltpu.SemaphoreType.DMA,
            pltpu.SemaphoreType.DMA,
            pltpu.SemaphoreType.DMA,
        ],
        compiler_params=pltpu.CompilerParams(use_tc_tiling_on_sc=False),
    )


MAIN = 999424
TAILB = 5
VOCABP = MAIN + TAILB * 128


MBS = 16384


def _relayout_main_kernel(t_ref, o_ref):
    tr = jnp.swapaxes(t_ref[...], 0, 1)
    o_ref[...] = jnp.concatenate(
        [tr[0:MBS // 2, :], tr[MBS // 2:MBS, :]], axis=1)


def _relayout_tail_kernel(alias_ref, t_ref, o_ref):
    del alias_ref
    tr = jnp.swapaxes(t_ref[...], 0, 1)
    o_ref[...] = jnp.concatenate([tr[0:64, :], tr[64:128, :]], axis=1)


def _relayout(table_t):
    main = pl.pallas_call(
        _relayout_main_kernel,
        grid=(MAIN // MBS,),
        in_specs=[pl.BlockSpec((DIM, MBS), lambda i: (0, i))],
        out_specs=pl.BlockSpec((MBS // 2, 2 * DIM), lambda i: (i, 0)),
        out_shape=jax.ShapeDtypeStruct((VOCABP // 2, 2 * DIM), jnp.float32),
    )(table_t)
    full = pl.pallas_call(
        _relayout_tail_kernel,
        grid=(TAILB,),
        in_specs=[
            pl.BlockSpec(memory_space=pltpu.MemorySpace.HBM),
            pl.BlockSpec((DIM, 128), lambda i: (0, MAIN // 128 + i)),
        ],
        out_specs=pl.BlockSpec((64, 2 * DIM), lambda i: (MAIN // 128 + i, 0)),
        out_shape=jax.ShapeDtypeStruct((VOCABP // 2, 2 * DIM), jnp.float32),
        input_output_aliases={0: 0},
    )(main, table_t)
    return full


def _mlp_kernel(sums_ref, len_ref, w1_ref, b1_ref, w2_ref, b2_ref, w3_ref,
                b3_ref, out_ref):
    rep = sums_ref[...] / len_ref[...]
    h1 = jnp.tanh(
        jnp.dot(rep, w1_ref[...], preferred_element_type=jnp.float32)
        + b1_ref[...]
    )
    z2 = (
        jnp.dot(h1, w2_ref[...], preferred_element_type=jnp.float32)
        + b2_ref[...]
    )
    h2 = jnp.where(z2 >= 0, z2, 0.01 * z2)
    out_ref[...] = (
        jnp.dot(h2, w3_ref[...], preferred_element_type=jnp.float32)
        + b3_ref[...]
    )


def _mlp(sums, lens_f, W1, b1, W2, b2, W3, b3):
    bs = 2048
    grid = B // bs
    full = lambda shape: pl.BlockSpec(shape, lambda i: (0, 0))
    return pl.pallas_call(
        _mlp_kernel,
        grid=(grid,),
        in_specs=[
            pl.BlockSpec((bs, DIM), lambda i: (i, 0)),
            pl.BlockSpec((bs, 1), lambda i: (i, 0)),
            full((DIM, H1)),
            full((1, H1)),
            full((H1, H2)),
            full((1, H2)),
            full((H2, OUT)),
            full((1, OUT)),
        ],
        out_specs=pl.BlockSpec((bs, OUT), lambda i: (i, 0)),
        out_shape=jax.ShapeDtypeStruct((B, OUT), jnp.float32),
    )(sums, lens_f, W1, b1, W2, b2, W3, b3)


@jax.jit
def kernel(x, lengths, table, W1, b1, W2, b2, W3, b3):
    xf = x.reshape(B * L)
    table_lin = _relayout(table.T).reshape(VOCABP, DIM)
    sums = _make_pool()(xf, table_lin)
    lens_f = lengths.astype(jnp.float32).reshape(B, 1)
    return _mlp(
        sums, lens_f, W1,
        b1.reshape(1, H1), W2, b2.reshape(1, H2), W3, b3.reshape(1, OUT),
    )
